# Initial kernel scaffold; baseline (speedup 1.0000x reference)
import jax
import jax.numpy as jnp
from jax.experimental import pallas as pl


def kernel(x, memory, memory_label):
    raise NotImplementedError("write your pallas kernel here")


# speedup vs baseline: 6.2195x; 6.2195x over previous
import functools
import math

import jax
import jax.numpy as jnp
from jax import lax
from jax.experimental import pallas as pl
from jax.experimental.pallas import tpu as pltpu
from jax.experimental.pallas import tpu_sc as plsc

B = 1024
DIM = 16
Q = 100000
TEMP = 0.07
K = 20
CLASSES = 1000

G = 128
NG = 784
QP = NG * G
CHUNK = 2048
GPC = CHUNK // G
NSTEP = QP // CHUNK
NCAND = 24
OUTP = 1008
NEG = -3.0e38

NW = 32
ROWS_PER_W = B // NW
NVREG = (K * G) // 16
SURV = K * G + 16


def _dist_body(x_ref, mem_ref, dist_ref, gmax_ref):
    xr = x_ref[...]
    nrm = jnp.sqrt(jnp.sum(xr * xr, axis=1, keepdims=True))
    xn = xr / jnp.maximum(nrm, 1e-12)
    d = lax.dot_general(
        xn, mem_ref[...],
        dimension_numbers=(((1,), (1,)), ((), ())),
        preferred_element_type=jnp.float32,
    )
    step = pl.program_id(0)
    col = lax.broadcasted_iota(jnp.int32, (B, CHUNK), 1) + step * CHUNK
    d = jnp.where(col < Q, d, NEG)
    dist_ref[...] = d
    gmax_ref[...] = jnp.max(d.reshape(B, GPC, G), axis=2).reshape(1, B, GPC)


def _dist_call(x, mem_p):
    return pl.pallas_call(
        _dist_body,
        grid=(NSTEP,),
        in_specs=[
            pl.BlockSpec((B, DIM), lambda i: (0, 0)),
            pl.BlockSpec((CHUNK, DIM), lambda i: (i, 0)),
        ],
        out_specs=[
            pl.BlockSpec((B, CHUNK), lambda i: (0, i)),
            pl.BlockSpec((1, B, GPC), lambda i: (i, 0, 0)),
        ],
        out_shape=[
            jax.ShapeDtypeStruct((B, QP), jnp.float32),
            jax.ShapeDtypeStruct((NSTEP, B, GPC), jnp.float32),
        ],
    )(x, mem_p)


def _groups_body(gmax_ref, gidx_ref, thr_ref):
    g = gmax_ref[...]
    colio = lax.broadcasted_iota(jnp.int32, (B, NG), 1)
    cols = []
    for k in range(K):
        m = jnp.max(g, axis=1, keepdims=True)
        ik = jnp.min(jnp.where(g == m, colio, jnp.int32(1 << 30)),
                     axis=1, keepdims=True)
        cols.append(ik)
        if k == K - 1:
            thr_ref[...] = jnp.broadcast_to(m, (B, 16))
        else:
            g = jnp.where(colio == ik, NEG, g)
    rowio = lax.broadcasted_iota(jnp.int32, (B, 1), 0)
    for j in range(NCAND - K):
        cols.append((rowio * 7 + j * 193) % (NG - 3))
    gidx_ref[...] = jnp.concatenate(cols, axis=1).astype(jnp.int32)


def _groups_call(gmax):
    return pl.pallas_call(
        _groups_body,
        out_shape=[
            jax.ShapeDtypeStruct((B, NCAND), jnp.int32),
            jax.ShapeDtypeStruct((B, 16), jnp.float32),
        ],
    )(gmax)


def _sc_body(dist2d, lab2d, gidx, thr, out,
             gx, idxv, thrv, cand, labv, sv, spos, outrow, sem1, sem2):
    cid = lax.axis_index("c")
    sid = lax.axis_index("s")
    wid = sid * 2 + cid
    lane = lax.iota(jnp.int32, 16)
    zeros16 = jnp.zeros((16,), jnp.float32)
    negs16 = jnp.full((16,), NEG, jnp.float32)
    izeros16 = jnp.zeros((16,), jnp.int32)

    def row_body(t, carry_unused):
        b = wid * ROWS_PER_W + t
        pltpu.sync_copy(gidx.at[b], gx)
        pltpu.sync_copy(thr.at[b], thrv)
        base = b * NG
        idxv[pl.ds(0, 16)] = gx[pl.ds(0, 16)] + base
        idxv[pl.ds(8, 16)] = gx[pl.ds(8, 16)] + base
        cp1 = pltpu.async_copy(dist2d.at[idxv], cand, sem1)
        cp2 = pltpu.async_copy(lab2d.at[gx], labv, sem2)
        cp1.wait()
        cp2.wait()
        thr_s = thrv[...][0]

        def pre(i, c):
            sv[pl.ds(i * 16, 16)] = negs16
            return c
        lax.fori_loop(0, SURV // 16, pre, 0)

        def comp(i, off):
            r = i >> 3
            c = i & 7
            v = cand[r, pl.ds(c * 16, 16)]
            m = v >= thr_s
            cnt = jnp.sum(m.astype(jnp.int32))
            pvec = i * 16 + lane
            plsc.store_compressed(sv.at[pl.ds(off, 16)], v, mask=m)
            plsc.store_compressed(spos.at[pl.ds(off, 16)], pvec, mask=m)
            return off + cnt
        s_cnt = lax.fori_loop(0, NVREG, comp, jnp.int32(0))
        nv = (s_cnt + 15) >> 4

        def sel(k, wcarry):
            wv0, wv1, wq0, wq1 = wcarry

            def scan(j, sc_carry):
                bv, bp = sc_carry
                v = sv[pl.ds(j * 16, 16)]
                p = j * 16 + lane
                m = v > bv
                return jnp.where(m, v, bv), jnp.where(m, p, bp)
            bv, bp = lax.fori_loop(0, nv, scan, (negs16, izeros16))
            sk, sp = plsc.sort_key_val(bv, bp, descending=True)
            wv = sk[0]
            wp = sp[0]
            m0 = lane == k
            m1 = lane == (k - 16)
            wv0 = jnp.where(m0, wv, wv0)
            wv1 = jnp.where(m1, wv, wv1)
            wq0 = jnp.where(m0, wp, wq0)
            wq1 = jnp.where(m1, wp, wq1)
            plsc.store_scatter(sv, [jnp.full((16,), wp, jnp.int32)],
                               negs16, mask=lane == 0)
            return wv0, wv1, wq0, wq1
        wv0, wv1, wq0, wq1 = lax.fori_loop(
            0, K, sel, (negs16, negs16, izeros16, izeros16))

        p0 = plsc.load_gather(spos, [wq0])
        p1 = plsc.load_gather(spos, [wq1])
        l0 = plsc.load_gather(labv, [p0 >> 7, p0 & (G - 1)])
        l1 = plsc.load_gather(labv, [p1 >> 7, p1 & (G - 1)])

        mm = jnp.maximum(jnp.max(wv0), jnp.max(wv1))
        inv_t = jnp.float32(1.0 / TEMP)
        a0 = jnp.maximum((wv0 - mm) * inv_t, -88.0)
        a1 = jnp.maximum((wv1 - mm) * inv_t, -88.0)
        e0 = jnp.exp(a0)
        e1 = jnp.exp(a1)
        ssum = jnp.full((16,), jnp.sum(e0) + jnp.sum(e1), jnp.float32)
        w0 = e0 / ssum
        w1 = e1 / ssum

        def zr(i, c):
            outrow[pl.ds(i * 16, 16)] = zeros16
            return c
        lax.fori_loop(0, OUTP // 16, zr, 0)
        onelane = lane == 0
        for k2 in range(K):
            lb = l0[k2] if k2 < 16 else l1[k2 - 16]
            wk = w0[k2] if k2 < 16 else w1[k2 - 16]
            plsc.addupdate_scatter(outrow, [jnp.full((16,), lb, jnp.int32)],
                                   jnp.full((16,), wk, jnp.float32),
                                   mask=onelane)

        def cl(i, c):
            u = outrow[pl.ds(i * 16, 16)]
            outrow[pl.ds(i * 16, 16)] = jnp.minimum(u + 1e-5, 1.0)
            return c
        lax.fori_loop(0, OUTP // 16, cl, 0)
        pltpu.sync_copy(outrow, out.at[b])
        return carry_unused

    lax.fori_loop(0, ROWS_PER_W, row_body, 0)


def _sc_call(dist2d, lab2d, gidx, thr):
    mesh = plsc.VectorSubcoreMesh(core_axis_name="c", subcore_axis_name="s")
    fn = functools.partial(
        pl.kernel,
        out_type=jax.ShapeDtypeStruct((B, OUTP), jnp.float32),
        mesh=mesh,
        compiler_params=pltpu.CompilerParams(needs_layout_passes=False),
        scratch_types=[
            pltpu.VMEM((NCAND,), jnp.int32),
            pltpu.VMEM((NCAND,), jnp.int32),
            pltpu.VMEM((16,), jnp.float32),
            pltpu.VMEM((NCAND, G), jnp.float32),
            pltpu.VMEM((NCAND, G), jnp.int32),
            pltpu.VMEM((SURV,), jnp.float32),
            pltpu.VMEM((SURV,), jnp.int32),
            pltpu.VMEM((OUTP,), jnp.float32),
            pltpu.SemaphoreType.DMA,
            pltpu.SemaphoreType.DMA,
        ],
    )(_sc_body)
    return fn(dist2d, lab2d, gidx, thr)


def kernel(x, memory, memory_label):
    mem_p = jnp.pad(memory, ((0, QP - Q), (0, 0)))
    lab_p = jnp.pad(memory_label, (0, QP - Q))
    dist, gmax3 = _dist_call(x, mem_p)
    gmax = jnp.transpose(gmax3, (1, 0, 2)).reshape(B, NG)
    gidx, thr = _groups_call(gmax)
    out = _sc_call(dist.reshape(B * NG, G), lab_p.reshape(NG, G), gidx, thr)
    return out[:, :CLASSES]


# speedup vs baseline: 9.7870x; 1.5736x over previous
import functools
import math

import jax
import jax.numpy as jnp
from jax import lax
from jax.experimental import pallas as pl
from jax.experimental.pallas import tpu as pltpu
from jax.experimental.pallas import tpu_sc as plsc

B = 1024
DIM = 16
Q = 100000
TEMP = 0.07
K = 20
CLASSES = 1000

G = 128
NG = 784
QP = NG * G
CHUNK = 2048
GPC = CHUNK // G
NSTEP = QP // CHUNK
NCAND = 24
OUTP = 1008
NEG = -3.0e38

NW = 32
ROWS_PER_W = B // NW
NVREG = (K * G) // 16
SURV = K * G + 16


def _dist_body(x_ref, mem_ref, dist_ref, gmax_ref):
    xr = x_ref[...]
    nrm = jnp.sqrt(jnp.sum(xr * xr, axis=1, keepdims=True))
    xn = xr / jnp.maximum(nrm, 1e-12)
    d = lax.dot_general(
        xn, mem_ref[...],
        dimension_numbers=(((1,), (1,)), ((), ())),
        preferred_element_type=jnp.float32,
    )
    step = pl.program_id(0)

    def _write(dm):
        parts = []
        for g in range(GPC):
            blk = dm[:, g * G:(g + 1) * G]
            dist_ref[g, :, :] = blk
            parts.append(jnp.max(blk, axis=1, keepdims=True))
        gmax_ref[...] = jnp.concatenate(parts, axis=1).reshape(1, B, GPC)

    @pl.when(step < NSTEP - 1)
    def _full():
        _write(d)

    @pl.when(step == NSTEP - 1)
    def _tail():
        col = lax.broadcasted_iota(jnp.int32, (B, CHUNK), 1) + step * CHUNK
        _write(jnp.where(col < Q, d, NEG))


def _dist_call(x, mem_p):
    return pl.pallas_call(
        _dist_body,
        grid=(NSTEP,),
        in_specs=[
            pl.BlockSpec((B, DIM), lambda i: (0, 0)),
            pl.BlockSpec((CHUNK, DIM), lambda i: (i, 0)),
        ],
        out_specs=[
            pl.BlockSpec((GPC, B, G), lambda i: (i, 0, 0)),
            pl.BlockSpec((1, B, GPC), lambda i: (i, 0, 0)),
        ],
        out_shape=[
            jax.ShapeDtypeStruct((NG, B, G), jnp.float32),
            jax.ShapeDtypeStruct((NSTEP, B, GPC), jnp.float32),
        ],
    )(x, mem_p)


def _groups_body(gmax_ref, gidx_ref, thr_ref):
    g = gmax_ref[...]
    colio = lax.broadcasted_iota(jnp.int32, (B, NG), 1)
    cols = []
    for k in range(K):
        m = jnp.max(g, axis=1, keepdims=True)
        ik = jnp.min(jnp.where(g == m, colio, jnp.int32(1 << 30)),
                     axis=1, keepdims=True)
        cols.append(ik)
        if k == K - 1:
            thr_ref[...] = jnp.broadcast_to(m, (B, 16))
        else:
            g = jnp.where(colio == ik, NEG, g)
    rowio = lax.broadcasted_iota(jnp.int32, (B, 1), 0)
    for j in range(NCAND - K):
        cols.append((rowio * 7 + j * 193) % (NG - 3))
    gidx_ref[...] = jnp.concatenate(cols, axis=1).astype(jnp.int32)


def _groups_call(gmax):
    return pl.pallas_call(
        _groups_body,
        out_shape=[
            jax.ShapeDtypeStruct((B, NCAND), jnp.int32),
            jax.ShapeDtypeStruct((B, 16), jnp.float32),
        ],
    )(gmax)


def _sc_body(dist2d, lab2d, gidx, thr, out,
             gx, idxv, thrv, cand, labv, sv, spos, outrow, sem1, sem2):
    cid = lax.axis_index("c")
    sid = lax.axis_index("s")
    wid = sid * 2 + cid
    lane = lax.iota(jnp.int32, 16)
    zeros16 = jnp.zeros((16,), jnp.float32)
    negs16 = jnp.full((16,), NEG, jnp.float32)
    izeros16 = jnp.zeros((16,), jnp.int32)

    def row_body(t, carry_unused):
        b = wid * ROWS_PER_W + t
        pltpu.sync_copy(gidx.at[b], gx)
        pltpu.sync_copy(thr.at[b], thrv)
        idxv[pl.ds(0, 16)] = gx[pl.ds(0, 16)] * B + b
        idxv[pl.ds(8, 16)] = gx[pl.ds(8, 16)] * B + b
        cp1 = pltpu.async_copy(dist2d.at[idxv], cand, sem1)
        cp2 = pltpu.async_copy(lab2d.at[gx], labv, sem2)
        cp1.wait()
        cp2.wait()
        thr_s = thrv[...][0]

        def pre(i, c):
            sv[pl.ds(i * 16, 16)] = negs16
            return c
        lax.fori_loop(0, SURV // 16, pre, 0)

        def comp(i, off):
            r = i >> 3
            c = i & 7
            v = cand[r, pl.ds(c * 16, 16)]
            m = v >= thr_s
            cnt = jnp.sum(m.astype(jnp.int32))
            pvec = i * 16 + lane
            plsc.store_compressed(sv.at[pl.ds(off, 16)], v, mask=m)
            plsc.store_compressed(spos.at[pl.ds(off, 16)], pvec, mask=m)
            return off + cnt
        s_cnt = lax.fori_loop(0, NVREG, comp, jnp.int32(0))
        nv = (s_cnt + 15) >> 4

        def sel(k, wcarry):
            wv0, wv1, wq0, wq1 = wcarry

            def scan(j, sc_carry):
                bv, bp = sc_carry
                v = sv[pl.ds(j * 16, 16)]
                p = j * 16 + lane
                m = v > bv
                return jnp.where(m, v, bv), jnp.where(m, p, bp)
            bv, bp = lax.fori_loop(0, nv, scan, (negs16, izeros16))
            sk, sp = plsc.sort_key_val(bv, bp, descending=True)
            wv = sk[0]
            wp = sp[0]
            m0 = lane == k
            m1 = lane == (k - 16)
            wv0 = jnp.where(m0, wv, wv0)
            wv1 = jnp.where(m1, wv, wv1)
            wq0 = jnp.where(m0, wp, wq0)
            wq1 = jnp.where(m1, wp, wq1)
            plsc.store_scatter(sv, [jnp.full((16,), wp, jnp.int32)],
                               negs16, mask=lane == 0)
            return wv0, wv1, wq0, wq1
        wv0, wv1, wq0, wq1 = lax.fori_loop(
            0, K, sel, (negs16, negs16, izeros16, izeros16))

        p0 = plsc.load_gather(spos, [wq0])
        p1 = plsc.load_gather(spos, [wq1])
        l0 = plsc.load_gather(labv, [p0 >> 7, p0 & (G - 1)])
        l1 = plsc.load_gather(labv, [p1 >> 7, p1 & (G - 1)])

        mm = jnp.maximum(jnp.max(wv0), jnp.max(wv1))
        inv_t = jnp.float32(1.0 / TEMP)
        a0 = jnp.maximum((wv0 - mm) * inv_t, -88.0)
        a1 = jnp.maximum((wv1 - mm) * inv_t, -88.0)
        e0 = jnp.exp(a0)
        e1 = jnp.exp(a1)
        ssum = jnp.full((16,), jnp.sum(e0) + jnp.sum(e1), jnp.float32)
        w0 = e0 / ssum
        w1 = e1 / ssum

        def zr(i, c):
            outrow[pl.ds(i * 16, 16)] = zeros16
            return c
        lax.fori_loop(0, OUTP // 16, zr, 0)
        onelane = lane == 0
        for k2 in range(K):
            lb = l0[k2] if k2 < 16 else l1[k2 - 16]
            wk = w0[k2] if k2 < 16 else w1[k2 - 16]
            plsc.addupdate_scatter(outrow, [jnp.full((16,), lb, jnp.int32)],
                                   jnp.full((16,), wk, jnp.float32),
                                   mask=onelane)

        def cl(i, c):
            u = outrow[pl.ds(i * 16, 16)]
            outrow[pl.ds(i * 16, 16)] = jnp.minimum(u + 1e-5, 1.0)
            return c
        lax.fori_loop(0, OUTP // 16, cl, 0)
        pltpu.sync_copy(outrow, out.at[b])
        return carry_unused

    lax.fori_loop(0, ROWS_PER_W, row_body, 0)


def _sc_call(dist2d, lab2d, gidx, thr):
    mesh = plsc.VectorSubcoreMesh(core_axis_name="c", subcore_axis_name="s")
    fn = functools.partial(
        pl.kernel,
        out_type=jax.ShapeDtypeStruct((B, OUTP), jnp.float32),
        mesh=mesh,
        compiler_params=pltpu.CompilerParams(needs_layout_passes=False),
        scratch_types=[
            pltpu.VMEM((NCAND,), jnp.int32),
            pltpu.VMEM((NCAND,), jnp.int32),
            pltpu.VMEM((16,), jnp.float32),
            pltpu.VMEM((NCAND, G), jnp.float32),
            pltpu.VMEM((NCAND, G), jnp.int32),
            pltpu.VMEM((SURV,), jnp.float32),
            pltpu.VMEM((SURV,), jnp.int32),
            pltpu.VMEM((OUTP,), jnp.float32),
            pltpu.SemaphoreType.DMA,
            pltpu.SemaphoreType.DMA,
        ],
    )(_sc_body)
    return fn(dist2d, lab2d, gidx, thr)


def kernel(x, memory, memory_label):
    lab_p = jnp.pad(memory_label, (0, QP - Q))
    dist, gmax3 = _dist_call(x, memory)
    gmax = jnp.transpose(gmax3, (1, 0, 2)).reshape(B, NG)
    gidx, thr = _groups_call(gmax)
    out = _sc_call(dist.reshape(NG * B, G), lab_p.reshape(NG, G), gidx, thr)
    return out[:, :CLASSES]


# speedup vs baseline: 11.4958x; 1.1746x over previous
import functools
import math

import jax
import jax.numpy as jnp
from jax import lax
from jax.experimental import pallas as pl
from jax.experimental.pallas import tpu as pltpu
from jax.experimental.pallas import tpu_sc as plsc

B = 1024
DIM = 16
Q = 100000
TEMP = 0.07
K = 20
CLASSES = 1000

G = 128
NG = 784
QP = NG * G
CHUNK = 2048
GPC = CHUNK // G
NSTEP = QP // CHUNK
NCAND = 24
OUTP = 1008
NEG = -3.0e38

NW = 32
ROWS_PER_W = B // NW
NVREG = (K * G) // 16
SURV = K * G + 16


def _dist_body(x_ref, mem_ref, dist_ref, gmax_ref):
    xr = x_ref[...]
    nrm = jnp.sqrt(jnp.sum(xr * xr, axis=1, keepdims=True))
    xn = xr / jnp.maximum(nrm, 1e-12)
    step = pl.program_id(0)

    def _write(mask_tail):
        parts = []
        for g in range(GPC):
            mg = mem_ref[pl.ds(g * G, G), :]
            dg = lax.dot_general(
                xn, mg,
                dimension_numbers=(((1,), (1,)), ((), ())),
                preferred_element_type=jnp.float32,
            )
            if mask_tail:
                col = (lax.broadcasted_iota(jnp.int32, (B, G), 1)
                       + step * CHUNK + g * G)
                dg = jnp.where(col < Q, dg, NEG)
            dist_ref[g, :, :] = dg
            parts.append(jnp.max(dg, axis=1, keepdims=True))
        gmax_ref[...] = jnp.concatenate(parts, axis=1).reshape(1, B, GPC)

    @pl.when(step < NSTEP - 1)
    def _full():
        _write(False)

    @pl.when(step == NSTEP - 1)
    def _tail():
        _write(True)


def _dist_call(x, mem_p):
    return pl.pallas_call(
        _dist_body,
        grid=(NSTEP,),
        in_specs=[
            pl.BlockSpec((B, DIM), lambda i: (0, 0)),
            pl.BlockSpec((CHUNK, DIM), lambda i: (i, 0)),
        ],
        out_specs=[
            pl.BlockSpec((GPC, B, G), lambda i: (i, 0, 0)),
            pl.BlockSpec((1, B, GPC), lambda i: (i, 0, 0)),
        ],
        out_shape=[
            jax.ShapeDtypeStruct((NG, B, G), jnp.float32),
            jax.ShapeDtypeStruct((NSTEP, B, GPC), jnp.float32),
        ],
    )(x, mem_p)


def _groups_body(gmax_ref, gidx_ref, thr_ref):
    g = gmax_ref[...]
    colio = lax.broadcasted_iota(jnp.int32, (B, NG), 1)
    cols = []
    for k in range(K):
        m = jnp.max(g, axis=1, keepdims=True)
        ik = jnp.min(jnp.where(g == m, colio, jnp.int32(1 << 30)),
                     axis=1, keepdims=True)
        cols.append(ik)
        if k == K - 1:
            thr_ref[...] = jnp.broadcast_to(m, (B, 16))
        else:
            g = jnp.where(colio == ik, NEG, g)
    rowio = lax.broadcasted_iota(jnp.int32, (B, 1), 0)
    for j in range(NCAND - K):
        cols.append((rowio * 7 + j * 193) % (NG - 3))
    gidx_ref[...] = jnp.concatenate(cols, axis=1).astype(jnp.int32)


def _groups_call(gmax):
    return pl.pallas_call(
        _groups_body,
        out_shape=[
            jax.ShapeDtypeStruct((B, NCAND), jnp.int32),
            jax.ShapeDtypeStruct((B, 16), jnp.float32),
        ],
    )(gmax)


def _sc_body(dist2d, lab2d, gidx, thr, out,
             gx, idxv, thrv, cand, labv, sv, slab, outrow, sem1, sem2):
    cid = lax.axis_index("c")
    sid = lax.axis_index("s")
    wid = sid * 2 + cid
    lane = lax.iota(jnp.int32, 16)
    zeros16 = jnp.zeros((16,), jnp.float32)
    negs16 = jnp.full((16,), NEG, jnp.float32)
    izeros16 = jnp.zeros((16,), jnp.int32)

    def row_body(t, carry_unused):
        b = wid * ROWS_PER_W + t
        pltpu.sync_copy(gidx.at[b], gx)
        pltpu.sync_copy(thr.at[b], thrv)
        idxv[pl.ds(0, 16)] = gx[pl.ds(0, 16)] * B + b
        idxv[pl.ds(8, 16)] = gx[pl.ds(8, 16)] * B + b
        cp1 = pltpu.async_copy(dist2d.at[idxv], cand, sem1)
        cp2 = pltpu.async_copy(lab2d.at[gx], labv, sem2)
        cp1.wait()
        cp2.wait()
        thr_s = thrv[...][0]

        def comp(i, off):
            r = i >> 3
            c = i & 7
            v = cand[r, pl.ds(c * 16, 16)]
            lv = labv[r, pl.ds(c * 16, 16)]
            m = v >= thr_s
            cnt = plsc.all_reduce_population_count(m)[0]
            plsc.store_compressed(sv.at[pl.ds(off, 16)], v, mask=m)
            plsc.store_compressed(slab.at[pl.ds(off, 16)], lv, mask=m)
            return off + cnt
        s_cnt = lax.fori_loop(0, NVREG, comp, jnp.int32(0))
        sv[pl.ds(s_cnt, 16)] = negs16
        nv = (s_cnt + 15) >> 4

        def sel(k, wcarry):
            wv0, wv1, wq0, wq1 = wcarry

            def scan(j, sc_carry):
                bv, bp = sc_carry
                v = sv[pl.ds(j * 16, 16)]
                p = j * 16 + lane
                m = v > bv
                return jnp.where(m, v, bv), jnp.where(m, p, bp)
            bv, bp = lax.fori_loop(0, nv, scan, (negs16, izeros16))
            sk, sp = plsc.sort_key_val(bv, bp, descending=True)
            wv = sk[0]
            wp = sp[0]
            m0 = lane == k
            m1 = lane == (k - 16)
            wv0 = jnp.where(m0, wv, wv0)
            wv1 = jnp.where(m1, wv, wv1)
            wq0 = jnp.where(m0, wp, wq0)
            wq1 = jnp.where(m1, wp, wq1)
            plsc.store_scatter(sv, [jnp.full((16,), wp, jnp.int32)],
                               negs16, mask=lane == 0)
            return wv0, wv1, wq0, wq1
        wv0, wv1, wq0, wq1 = lax.fori_loop(
            0, K, sel, (negs16, negs16, izeros16, izeros16))

        l0 = plsc.load_gather(slab, [wq0])
        l1 = plsc.load_gather(slab, [wq1])

        mm = jnp.maximum(jnp.max(wv0), jnp.max(wv1))
        inv_t = jnp.float32(1.0 / TEMP)
        a0 = jnp.maximum((wv0 - mm) * inv_t, -88.0)
        a1 = jnp.maximum((wv1 - mm) * inv_t, -88.0)
        e0 = jnp.exp(a0)
        e1 = jnp.exp(a1)
        ssum = jnp.full((16,), jnp.sum(e0) + jnp.sum(e1), jnp.float32)
        w0 = e0 / ssum
        w1 = e1 / ssum

        def zr(i, c):
            outrow[pl.ds(i * 16, 16)] = zeros16
            return c
        lax.fori_loop(0, OUTP // 16, zr, 0)
        onelane = lane == 0
        for k2 in range(K):
            lb = l0[k2] if k2 < 16 else l1[k2 - 16]
            wk = w0[k2] if k2 < 16 else w1[k2 - 16]
            plsc.addupdate_scatter(outrow, [jnp.full((16,), lb, jnp.int32)],
                                   jnp.full((16,), wk, jnp.float32),
                                   mask=onelane)

        def cl(i, c):
            u = outrow[pl.ds(i * 16, 16)]
            outrow[pl.ds(i * 16, 16)] = jnp.minimum(u + 1e-5, 1.0)
            return c
        lax.fori_loop(0, OUTP // 16, cl, 0)
        pltpu.sync_copy(outrow, out.at[b])
        return carry_unused

    lax.fori_loop(0, ROWS_PER_W, row_body, 0)


def _sc_call(dist2d, lab2d, gidx, thr):
    mesh = plsc.VectorSubcoreMesh(core_axis_name="c", subcore_axis_name="s")
    fn = functools.partial(
        pl.kernel,
        out_type=jax.ShapeDtypeStruct((B, OUTP), jnp.float32),
        mesh=mesh,
        compiler_params=pltpu.CompilerParams(needs_layout_passes=False),
        scratch_types=[
            pltpu.VMEM((NCAND,), jnp.int32),
            pltpu.VMEM((NCAND,), jnp.int32),
            pltpu.VMEM((16,), jnp.float32),
            pltpu.VMEM((NCAND, G), jnp.float32),
            pltpu.VMEM((NCAND, G), jnp.int32),
            pltpu.VMEM((SURV,), jnp.float32),
            pltpu.VMEM((SURV,), jnp.int32),
            pltpu.VMEM((OUTP,), jnp.float32),
            pltpu.SemaphoreType.DMA,
            pltpu.SemaphoreType.DMA,
        ],
    )(_sc_body)
    return fn(dist2d, lab2d, gidx, thr)


def kernel(x, memory, memory_label):
    lab_p = jnp.pad(memory_label, (0, QP - Q))
    dist, gmax3 = _dist_call(x, memory)
    gmax = jnp.transpose(gmax3, (1, 0, 2)).reshape(B, NG)
    gidx, thr = _groups_call(gmax)
    out = _sc_call(dist.reshape(NG * B, G), lab_p.reshape(NG, G), gidx, thr)
    return out[:, :CLASSES]


# speedup vs baseline: 14.0911x; 1.2258x over previous
import functools
import math

import jax
import jax.numpy as jnp
from jax import lax
from jax.experimental import pallas as pl
from jax.experimental.pallas import tpu as pltpu
from jax.experimental.pallas import tpu_sc as plsc

B = 1024
DIM = 16
Q = 100000
TEMP = 0.07
K = 20
CLASSES = 1000

G = 128
NG = 784
QP = NG * G
CHUNK = 2048
GPC = CHUNK // G
NSTEP = QP // CHUNK
NCAND = 24
OUTP = 1008
NEG = -3.0e38

NW = 32
ROWS_PER_W = B // NW
NVREG = (K * G) // 16
SURV = K * G + 16


def _dist_body(x_ref, mem_ref, dist_ref, gmax_ref):
    xr = x_ref[...]
    nrm = jnp.sqrt(jnp.sum(xr * xr, axis=1, keepdims=True))
    xn = xr / jnp.maximum(nrm, 1e-12)
    step = pl.program_id(0)

    def _write(mask_tail):
        parts = []
        for g in range(GPC):
            mg = mem_ref[:, pl.ds(g * G, G)]
            dg = lax.dot_general(
                xn, mg,
                dimension_numbers=(((1,), (0,)), ((), ())),
                preferred_element_type=jnp.float32,
            )
            if mask_tail:
                col = (lax.broadcasted_iota(jnp.int32, (B, G), 1)
                       + step * CHUNK + g * G)
                dg = jnp.where(col < Q, dg, NEG)
            dist_ref[g, :, :] = dg
            parts.append(jnp.max(dg, axis=1, keepdims=True))
        gmax_ref[...] = jnp.concatenate(parts, axis=1).reshape(1, B, GPC)

    @pl.when(step < NSTEP - 1)
    def _full():
        _write(False)

    @pl.when(step == NSTEP - 1)
    def _tail():
        _write(True)


def _dist_call(x, mem_p):
    return pl.pallas_call(
        _dist_body,
        grid=(NSTEP,),
        in_specs=[
            pl.BlockSpec((B, DIM), lambda i: (0, 0)),
            pl.BlockSpec((DIM, CHUNK), lambda i: (0, i)),
        ],
        out_specs=[
            pl.BlockSpec((GPC, B, G), lambda i: (i, 0, 0)),
            pl.BlockSpec((1, B, GPC), lambda i: (i, 0, 0)),
        ],
        out_shape=[
            jax.ShapeDtypeStruct((NG, B, G), jnp.float32),
            jax.ShapeDtypeStruct((NSTEP, B, GPC), jnp.float32),
        ],
    )(x, mem_p)


def _groups_body(gmax_ref, gt_ref):
    g = gmax_ref[...]
    colio = lax.broadcasted_iota(jnp.int32, (B, NG), 1)
    cols = []
    thr_bits = None
    for k in range(K):
        m = jnp.max(g, axis=1, keepdims=True)
        ik = jnp.min(jnp.where(g == m, colio, jnp.int32(1 << 30)),
                     axis=1, keepdims=True)
        cols.append(ik)
        if k == K - 1:
            thr_bits = lax.bitcast_convert_type(m, jnp.int32)
        else:
            g = jnp.where(colio == ik, NEG, g)
    rowio = lax.broadcasted_iota(jnp.int32, (B, 1), 0)
    for j in range(NCAND - K):
        cols.append((rowio * 7 + j * 193) % (NG - 3))
    for j in range(8):
        cols.append(thr_bits)
    gt_ref[...] = jnp.concatenate(cols, axis=1).astype(jnp.int32)


def _groups_call(gmax):
    return pl.pallas_call(
        _groups_body,
        out_shape=jax.ShapeDtypeStruct((B, 32), jnp.int32),
    )(gmax)


def _sc_body(dist2d, lab2d, gt, out,
             gx0, gx1, idx0, idx1, lidx0, lidx1,
             cand0, cand1, labv0, labv1, sv, slab, outrow,
             semd0, semd1, seml0, seml1):
    cid = lax.axis_index("c")
    sid = lax.axis_index("s")
    wid = sid * 2 + cid
    lane = lax.iota(jnp.int32, 16)
    zeros16 = jnp.zeros((16,), jnp.float32)
    negs16 = jnp.full((16,), NEG, jnp.float32)
    izeros16 = jnp.zeros((16,), jnp.int32)
    row0 = wid * ROWS_PER_W

    def fetch(b, gxb, idxb, lidxb, candb, labvb, semd, seml):
        pltpu.sync_copy(gt.at[b], gxb)
        g0 = gxb[pl.ds(0, 16)]
        g1 = gxb[pl.ds(8, 16)]
        idxb[pl.ds(0, 16)] = g0 * B + b
        idxb[pl.ds(8, 16)] = g1 * B + b
        lidxb[pl.ds(0, 16)] = g0
        lidxb[pl.ds(8, 16)] = g1
        pltpu.async_copy(dist2d.at[idxb], candb, semd)
        pltpu.async_copy(lab2d.at[lidxb], labvb, seml)

    def compute(b, gxb, idxb, lidxb, candb, labvb, semd, seml):
        pltpu.make_async_copy(dist2d.at[idxb], candb, semd).wait()
        pltpu.make_async_copy(lab2d.at[lidxb], labvb, seml).wait()
        thr_s = plsc.bitcast(gxb[pl.ds(16, 16)], jnp.float32)[8]

        def comp(i, off):
            r = i >> 3
            c = i & 7
            v = candb[r, pl.ds(c * 16, 16)]
            lv = labvb[r, pl.ds(c * 16, 16)]
            m = v >= thr_s
            cnt = plsc.all_reduce_population_count(m)[0]
            plsc.store_compressed(sv.at[pl.ds(off, 16)], v, mask=m)
            plsc.store_compressed(slab.at[pl.ds(off, 16)], lv, mask=m)
            return off + cnt
        s_cnt = lax.fori_loop(0, NVREG, comp, jnp.int32(0))
        sv[pl.ds(s_cnt, 16)] = negs16
        nv = (s_cnt + 15) >> 4

        def sel(k, wcarry):
            wv0, wv1, wq0, wq1 = wcarry

            def scan(j, sc_carry):
                bv, bp = sc_carry
                v = sv[pl.ds(j * 16, 16)]
                pv = j * 16 + lane
                m = v > bv
                return jnp.where(m, v, bv), jnp.where(m, pv, bp)
            bv, bp = lax.fori_loop(0, nv, scan, (negs16, izeros16))
            sk, sp = plsc.sort_key_val(bv, bp, descending=True)
            wv = sk[0]
            wp = sp[0]
            m0 = lane == k
            m1 = lane == (k - 16)
            wv0 = jnp.where(m0, wv, wv0)
            wv1 = jnp.where(m1, wv, wv1)
            wq0 = jnp.where(m0, wp, wq0)
            wq1 = jnp.where(m1, wp, wq1)
            plsc.store_scatter(sv, [jnp.full((16,), wp, jnp.int32)],
                               negs16, mask=lane == 0)
            return wv0, wv1, wq0, wq1
        wv0, wv1, wq0, wq1 = lax.fori_loop(
            0, K, sel, (negs16, negs16, izeros16, izeros16))

        l0 = plsc.load_gather(slab, [wq0])
        l1 = plsc.load_gather(slab, [wq1])

        mm = jnp.maximum(jnp.max(wv0), jnp.max(wv1))
        inv_t = jnp.float32(1.0 / TEMP)
        a0 = jnp.maximum((wv0 - mm) * inv_t, -88.0)
        a1 = jnp.maximum((wv1 - mm) * inv_t, -88.0)
        e0 = jnp.exp(a0)
        e1 = jnp.exp(a1)
        ssum = jnp.full((16,), jnp.sum(e0) + jnp.sum(e1), jnp.float32)
        w0 = e0 / ssum
        w1 = e1 / ssum

        def zr(i, c):
            outrow[pl.ds(i * 16, 16)] = zeros16
            return c
        lax.fori_loop(0, OUTP // 16, zr, 0)
        onelane = lane == 0
        for k2 in range(K):
            lb = l0[k2] if k2 < 16 else l1[k2 - 16]
            wk = w0[k2] if k2 < 16 else w1[k2 - 16]
            plsc.addupdate_scatter(outrow, [jnp.full((16,), lb, jnp.int32)],
                                   jnp.full((16,), wk, jnp.float32),
                                   mask=onelane)

        def cl(i, c):
            u = outrow[pl.ds(i * 16, 16)]
            outrow[pl.ds(i * 16, 16)] = jnp.minimum(u + 1e-5, 1.0)
            return c
        lax.fori_loop(0, OUTP // 16, cl, 0)
        pltpu.sync_copy(outrow, out.at[b])

    fetch(row0, gx0, idx0, lidx0, cand0, labv0, semd0, seml0)

    def pair(i, c):
        b0 = row0 + i * 2
        fetch(b0 + 1, gx1, idx1, lidx1, cand1, labv1, semd1, seml1)
        compute(b0, gx0, idx0, lidx0, cand0, labv0, semd0, seml0)

        @pl.when(i < ROWS_PER_W // 2 - 1)
        def _():
            fetch(b0 + 2, gx0, idx0, lidx0, cand0, labv0, semd0, seml0)
        compute(b0 + 1, gx1, idx1, lidx1, cand1, labv1, semd1, seml1)
        return c

    lax.fori_loop(0, ROWS_PER_W // 2, pair, 0)


def _sc_call(dist2d, lab2d, gt):
    mesh = plsc.VectorSubcoreMesh(core_axis_name="c", subcore_axis_name="s")
    fn = functools.partial(
        pl.kernel,
        out_type=jax.ShapeDtypeStruct((B, OUTP), jnp.float32),
        mesh=mesh,
        compiler_params=pltpu.CompilerParams(needs_layout_passes=False),
        scratch_types=[
            pltpu.VMEM((32,), jnp.int32),
            pltpu.VMEM((32,), jnp.int32),
            pltpu.VMEM((NCAND,), jnp.int32),
            pltpu.VMEM((NCAND,), jnp.int32),
            pltpu.VMEM((NCAND,), jnp.int32),
            pltpu.VMEM((NCAND,), jnp.int32),
            pltpu.VMEM((NCAND, G), jnp.float32),
            pltpu.VMEM((NCAND, G), jnp.float32),
            pltpu.VMEM((NCAND, G), jnp.int32),
            pltpu.VMEM((NCAND, G), jnp.int32),
            pltpu.VMEM((SURV,), jnp.float32),
            pltpu.VMEM((SURV,), jnp.int32),
            pltpu.VMEM((OUTP,), jnp.float32),
            pltpu.SemaphoreType.DMA,
            pltpu.SemaphoreType.DMA,
            pltpu.SemaphoreType.DMA,
            pltpu.SemaphoreType.DMA,
        ],
    )(_sc_body)
    return fn(dist2d, lab2d, gt)


def kernel(x, memory, memory_label):
    lab_p = jnp.pad(memory_label, (0, QP - Q))
    dist, gmax3 = _dist_call(x, memory.T)
    gmax = jnp.transpose(gmax3, (1, 0, 2)).reshape(B, NG)
    gt = _groups_call(gmax)
    out = _sc_call(dist.reshape(NG * B, G), lab_p.reshape(NG, G), gt)
    return out[:, :CLASSES]


# speedup vs baseline: 14.8786x; 1.0559x over previous
import functools
import math

import jax
import jax.numpy as jnp
from jax import lax
from jax.experimental import pallas as pl
from jax.experimental.pallas import tpu as pltpu
from jax.experimental.pallas import tpu_sc as plsc

B = 1024
DIM = 16
Q = 100000
TEMP = 0.07
K = 20
CLASSES = 1000

G = 128
NG = 784
QP = NG * G
CHUNK = 2048
GPC = CHUNK // G
NSTEP = QP // CHUNK
NCAND = 24
OUTP = 1008
NEG = -3.0e38

NW = 32
ROWS_PER_W = B // NW
NVREG = (K * G) // 16
SURV = K * G + 16


NGP = 896


def _dist_body(x_ref, mem_ref, dist_ref, gt_ref, gm_ref):
    xr = x_ref[...]
    nrm = jnp.sqrt(jnp.sum(xr * xr, axis=1, keepdims=True))
    xn = xr / jnp.maximum(nrm, 1e-12)
    step = pl.program_id(0)

    def _write(mask_tail):
        parts = []
        for g in range(GPC):
            mg = mem_ref[:, pl.ds(g * G, G)]
            dg = lax.dot_general(
                xn, mg,
                dimension_numbers=(((1,), (0,)), ((), ())),
                preferred_element_type=jnp.float32,
            )
            if mask_tail:
                col = (lax.broadcasted_iota(jnp.int32, (B, G), 1)
                       + step * CHUNK + g * G)
                dg = jnp.where(col < Q, dg, NEG)
            dist_ref[g, :, :] = dg
            parts.append(jnp.max(dg, axis=1, keepdims=True))
        pc = jnp.concatenate(parts, axis=1)
        for r in range(8):
            @pl.when((step & 7) == r)
            def _(pc=pc, r=r):
                gm_ref[step >> 3, :, r * GPC:(r + 1) * GPC] = pc

    @pl.when(step < NSTEP - 1)
    def _full():
        _write(False)

    @pl.when(step == NSTEP - 1)
    def _tail():
        _write(True)
        gm_ref[(NSTEP - 1) >> 3, :, GPC:] = jnp.full((B, G - GPC), NEG,
                                                     jnp.float32)
        g = jnp.concatenate([gm_ref[i] for i in range(NGP // G)], axis=1)
        colio = lax.broadcasted_iota(jnp.int32, (B, NGP), 1)
        cols = []
        thr_bits = None
        for k in range(K):
            m = jnp.max(g, axis=1, keepdims=True)
            ik = jnp.min(jnp.where(g == m, colio, jnp.int32(1 << 30)),
                         axis=1, keepdims=True)
            cols.append(ik)
            if k == K - 1:
                thr_bits = lax.bitcast_convert_type(m, jnp.int32)
            else:
                g = jnp.where(colio == ik, NEG, g)
        rowio = lax.broadcasted_iota(jnp.int32, (B, 1), 0)
        for j in range(NCAND - K):
            cols.append((rowio * 7 + j * 193) % (NG - 3))
        for j in range(8):
            cols.append(thr_bits)
        gt_ref[...] = jnp.concatenate(cols, axis=1).astype(jnp.int32)


def _dist_call(x, mem_t):
    return pl.pallas_call(
        _dist_body,
        grid=(NSTEP,),
        in_specs=[
            pl.BlockSpec((B, DIM), lambda i: (0, 0)),
            pl.BlockSpec((DIM, CHUNK), lambda i: (0, i)),
        ],
        out_specs=[
            pl.BlockSpec((GPC, B, G), lambda i: (i, 0, 0)),
            pl.BlockSpec((B, 32), lambda i: (0, 0)),
        ],
        out_shape=[
            jax.ShapeDtypeStruct((NG, B, G), jnp.float32),
            jax.ShapeDtypeStruct((B, 32), jnp.int32),
        ],
        scratch_shapes=[pltpu.VMEM((NGP // G, B, G), jnp.float32)],
    )(x, mem_t)


def _sc_body(dist2d, lab2d, gt, out,
             gx0, gx1, idx0, idx1, lidx0, lidx1,
             cand0, cand1, labv0, labv1, sv, slab, outrow,
             semd0, semd1, seml0, seml1):
    cid = lax.axis_index("c")
    sid = lax.axis_index("s")
    wid = sid * 2 + cid
    lane = lax.iota(jnp.int32, 16)
    zeros16 = jnp.zeros((16,), jnp.float32)
    negs16 = jnp.full((16,), NEG, jnp.float32)
    izeros16 = jnp.zeros((16,), jnp.int32)
    row0 = wid * ROWS_PER_W

    def fetch(b, gxb, idxb, lidxb, candb, labvb, semd, seml):
        pltpu.sync_copy(gt.at[b], gxb)
        g0 = gxb[pl.ds(0, 16)]
        g1 = gxb[pl.ds(8, 16)]
        idxb[pl.ds(0, 16)] = g0 * B + b
        idxb[pl.ds(8, 16)] = g1 * B + b
        lidxb[pl.ds(0, 16)] = g0
        lidxb[pl.ds(8, 16)] = g1
        pltpu.async_copy(dist2d.at[idxb], candb, semd)
        pltpu.async_copy(lab2d.at[lidxb], labvb, seml)

    def compute(b, gxb, idxb, lidxb, candb, labvb, semd, seml):
        pltpu.make_async_copy(dist2d.at[idxb], candb, semd).wait()
        pltpu.make_async_copy(lab2d.at[lidxb], labvb, seml).wait()
        thr_s = plsc.bitcast(gxb[pl.ds(16, 16)], jnp.float32)[8]

        def comp(i, off):
            r = i >> 3
            c = i & 7
            v = candb[r, pl.ds(c * 16, 16)]
            lv = labvb[r, pl.ds(c * 16, 16)]
            m = v >= thr_s
            cnt = plsc.all_reduce_population_count(m)[0]
            plsc.store_compressed(sv.at[pl.ds(off, 16)], v, mask=m)
            plsc.store_compressed(slab.at[pl.ds(off, 16)], lv, mask=m)
            return off + cnt
        s_cnt = lax.fori_loop(0, NVREG, comp, jnp.int32(0))
        sv[pl.ds(s_cnt, 16)] = negs16
        nv = (s_cnt + 15) >> 4

        def sel(k, wcarry):
            wv0, wv1, wq0, wq1 = wcarry

            def scan(j, sc_carry):
                bv, bp = sc_carry
                v = sv[pl.ds(j * 16, 16)]
                pv = j * 16 + lane
                m = v > bv
                return jnp.where(m, v, bv), jnp.where(m, pv, bp)
            bv, bp = lax.fori_loop(0, nv, scan, (negs16, izeros16))
            sk, sp = plsc.sort_key_val(bv, bp, descending=True)
            wv = sk[0]
            wp = sp[0]
            m0 = lane == k
            m1 = lane == (k - 16)
            wv0 = jnp.where(m0, wv, wv0)
            wv1 = jnp.where(m1, wv, wv1)
            wq0 = jnp.where(m0, wp, wq0)
            wq1 = jnp.where(m1, wp, wq1)
            plsc.store_scatter(sv, [jnp.full((16,), wp, jnp.int32)],
                               negs16, mask=lane == 0)
            return wv0, wv1, wq0, wq1
        wv0, wv1, wq0, wq1 = lax.fori_loop(
            0, K, sel, (negs16, negs16, izeros16, izeros16))

        l0 = plsc.load_gather(slab, [wq0])
        l1 = plsc.load_gather(slab, [wq1])

        mm = jnp.maximum(jnp.max(wv0), jnp.max(wv1))
        inv_t = jnp.float32(1.0 / TEMP)
        a0 = jnp.maximum((wv0 - mm) * inv_t, -88.0)
        a1 = jnp.maximum((wv1 - mm) * inv_t, -88.0)
        e0 = jnp.exp(a0)
        e1 = jnp.exp(a1)
        ssum = jnp.full((16,), jnp.sum(e0) + jnp.sum(e1), jnp.float32)
        w0 = e0 / ssum
        w1 = e1 / ssum

        def zr(i, c):
            outrow[pl.ds(i * 16, 16)] = zeros16
            return c
        lax.fori_loop(0, OUTP // 16, zr, 0)
        onelane = lane == 0
        for k2 in range(K):
            lb = l0[k2] if k2 < 16 else l1[k2 - 16]
            wk = w0[k2] if k2 < 16 else w1[k2 - 16]
            plsc.addupdate_scatter(outrow, [jnp.full((16,), lb, jnp.int32)],
                                   jnp.full((16,), wk, jnp.float32),
                                   mask=onelane)

        def cl(i, c):
            u = outrow[pl.ds(i * 16, 16)]
            outrow[pl.ds(i * 16, 16)] = jnp.minimum(u + 1e-5, 1.0)
            return c
        lax.fori_loop(0, OUTP // 16, cl, 0)
        pltpu.sync_copy(outrow, out.at[b])

    fetch(row0, gx0, idx0, lidx0, cand0, labv0, semd0, seml0)

    def pair(i, c):
        b0 = row0 + i * 2
        fetch(b0 + 1, gx1, idx1, lidx1, cand1, labv1, semd1, seml1)
        compute(b0, gx0, idx0, lidx0, cand0, labv0, semd0, seml0)

        @pl.when(i < ROWS_PER_W // 2 - 1)
        def _():
            fetch(b0 + 2, gx0, idx0, lidx0, cand0, labv0, semd0, seml0)
        compute(b0 + 1, gx1, idx1, lidx1, cand1, labv1, semd1, seml1)
        return c

    lax.fori_loop(0, ROWS_PER_W // 2, pair, 0)


def _sc_call(dist2d, lab2d, gt):
    mesh = plsc.VectorSubcoreMesh(core_axis_name="c", subcore_axis_name="s")
    fn = functools.partial(
        pl.kernel,
        out_type=jax.ShapeDtypeStruct((B, OUTP), jnp.float32),
        mesh=mesh,
        compiler_params=pltpu.CompilerParams(needs_layout_passes=False),
        scratch_types=[
            pltpu.VMEM((32,), jnp.int32),
            pltpu.VMEM((32,), jnp.int32),
            pltpu.VMEM((NCAND,), jnp.int32),
            pltpu.VMEM((NCAND,), jnp.int32),
            pltpu.VMEM((NCAND,), jnp.int32),
            pltpu.VMEM((NCAND,), jnp.int32),
            pltpu.VMEM((NCAND, G), jnp.float32),
            pltpu.VMEM((NCAND, G), jnp.float32),
            pltpu.VMEM((NCAND, G), jnp.int32),
            pltpu.VMEM((NCAND, G), jnp.int32),
            pltpu.VMEM((SURV,), jnp.float32),
            pltpu.VMEM((SURV,), jnp.int32),
            pltpu.VMEM((OUTP,), jnp.float32),
            pltpu.SemaphoreType.DMA,
            pltpu.SemaphoreType.DMA,
            pltpu.SemaphoreType.DMA,
            pltpu.SemaphoreType.DMA,
        ],
    )(_sc_body)
    return fn(dist2d, lab2d, gt)


def kernel(x, memory, memory_label):
    lab_p = jnp.pad(memory_label, (0, QP - Q))
    dist, gt = _dist_call(x, memory.T)
    out = _sc_call(dist.reshape(NG * B, G), lab_p.reshape(NG, G), gt)
    return out[:, :CLASSES]
